# baseline (speedup 1.0000x reference)
import functools

import jax
import jax.numpy as jnp
from jax import lax
from jax.experimental import pallas as pl
from jax.experimental.pallas import tpu as pltpu
from jax.experimental.pallas import tpu_sc as plsc

_EMB = 128
_LPAD = 56
_TOT = 16384 * _LPAD
_NW = 32
_PER_W = _TOT // _NW
_G = 128
_NCH = _PER_W // _G


_NBUF = 4


def _make_gather():
    mesh = plsc.VectorSubcoreMesh(core_axis_name="c", subcore_axis_name="s")

    @functools.partial(
        pl.kernel,
        out_type=jax.ShapeDtypeStruct((_TOT, _EMB), jnp.float32),
        mesh=mesh,
        scratch_types=[
            pltpu.VMEM((_NCH, _G), jnp.int32),
            pltpu.VMEM((_NBUF, _G, _EMB), jnp.float32),
        ] + [pltpu.SemaphoreType.DMA] * (2 * _NBUF),
    )
    def gather_kernel(idx_hbm, table_hbm, out_hbm, idx_v, rows_v, *sems):
        gsem = sems[:_NBUF]
        ssem = sems[_NBUF:]
        wid = lax.axis_index("s") * 2 + lax.axis_index("c")
        base = wid * _PER_W
        pltpu.sync_copy(idx_hbm.at[wid], idx_v)

        def start_gather(j, b):
            pltpu.async_copy(table_hbm.at[idx_v.at[j]], rows_v.at[b], gsem[b])

        def start_store(j, b):
            pltpu.async_copy(
                rows_v.at[b], out_hbm.at[pl.ds(base + j * _G, _G)], ssem[b])

        def wait_gather(b):
            pltpu.make_async_copy(
                table_hbm.at[pl.ds(0, _G)], rows_v.at[b], gsem[b]).wait()

        def wait_store(j, b):
            pltpu.make_async_copy(
                rows_v.at[b], out_hbm.at[pl.ds(base + j * _G, _G)],
                ssem[b]).wait()

        for b in range(_NBUF):
            start_gather(b, b)

        def body(t, carry):
            j0 = t * _NBUF
            for b in range(_NBUF):
                wait_gather(b)
                start_store(j0 + b, b)
            for b in range(_NBUF):
                wait_store(j0 + b, b)
                start_gather(j0 + b + _NBUF, b)
            return carry

        lax.fori_loop(0, _NCH // _NBUF - 1, body, 0)

        j0 = _NCH - _NBUF
        for b in range(_NBUF):
            wait_gather(b)
            start_store(j0 + b, b)
        for b in range(_NBUF):
            wait_store(j0 + b, b)

    return gather_kernel


_gather = _make_gather()


def kernel(data, ivectors_weight):
    idx = jnp.pad(data.astype(jnp.int32), ((0, 0), (0, _LPAD - 50)))
    idx = idx.reshape(_NW, _NCH, _G)
    out = _gather(idx, ivectors_weight)
    return out.reshape(data.shape[0], _LPAD, _EMB)[:, :50, :]


# speedup vs baseline: 4.7984x; 4.7984x over previous
import functools

import jax
import jax.numpy as jnp
from jax import lax
from jax.experimental import pallas as pl
from jax.experimental.pallas import tpu as pltpu
from jax.experimental.pallas import tpu_sc as plsc

_BATCH = 16384
_L = 50
_EMB = 128
_TOT = _BATCH * _L
_NW = 32
_PER_W = _TOT // _NW
_G = 128
_NCH = _PER_W // _G
_NBUF = 4


def _make_gather():
    mesh = plsc.VectorSubcoreMesh(core_axis_name="c", subcore_axis_name="s")

    @functools.partial(
        pl.kernel,
        out_type=jax.ShapeDtypeStruct((_TOT, _EMB), jnp.float32),
        mesh=mesh,
        scratch_types=[
            pltpu.VMEM((_NCH, _G), jnp.int32),
            pltpu.VMEM((_NBUF, _G, _EMB), jnp.float32),
        ] + [pltpu.SemaphoreType.DMA] * (2 * _NBUF),
    )
    def gather_kernel(idx_hbm, table_hbm, out_hbm, idx_v, rows_v, *sems):
        gsem = sems[:_NBUF]
        ssem = sems[_NBUF:]
        wid = lax.axis_index("s") * 2 + lax.axis_index("c")
        base = wid * _PER_W
        pltpu.sync_copy(idx_hbm.at[wid], idx_v)

        def start_gather(j, b):
            pltpu.async_copy(table_hbm.at[idx_v.at[j]], rows_v.at[b], gsem[b])

        def start_store(j, b):
            pltpu.async_copy(
                rows_v.at[b], out_hbm.at[pl.ds(base + j * _G, _G)], ssem[b])

        def wait_gather(b):
            pltpu.make_async_copy(
                table_hbm.at[pl.ds(0, _G)], rows_v.at[b], gsem[b]).wait()

        def wait_store(j, b):
            pltpu.make_async_copy(
                rows_v.at[b], out_hbm.at[pl.ds(base + j * _G, _G)],
                ssem[b]).wait()

        for b in range(_NBUF):
            start_gather(b, b)

        def body(t, carry):
            j0 = t * _NBUF
            for b in range(_NBUF):
                wait_gather(b)
                start_store(j0 + b, b)
            for b in range(_NBUF):
                wait_store(j0 + b, b)
                start_gather(j0 + b + _NBUF, b)
            return carry

        lax.fori_loop(0, _NCH // _NBUF - 1, body, 0)

        j0 = _NCH - _NBUF
        for b in range(_NBUF):
            wait_gather(b)
            start_store(j0 + b, b)
        for b in range(_NBUF):
            wait_store(j0 + b, b)

    return gather_kernel


_gather = _make_gather()

_NB = 32


def _relayout_body(flat_ref, out_ref):
    for k in range(_NB):
        out_ref[k] = flat_ref[pl.ds(k * _L, _L), :]


_relayout = pl.pallas_call(
    _relayout_body,
    grid=(_BATCH // _NB,),
    in_specs=[pl.BlockSpec((_NB * _L, _EMB), lambda i: (i, 0))],
    out_specs=pl.BlockSpec((_NB, _L, _EMB), lambda i: (i, 0, 0)),
    out_shape=jax.ShapeDtypeStruct((_BATCH, _L, _EMB), jnp.float32),
)


def kernel(data, ivectors_weight):
    idx = data.reshape(_NW, _NCH, _G).astype(jnp.int32)
    flat = _gather(idx, ivectors_weight)
    return _relayout(flat)


# speedup vs baseline: 15.8154x; 3.2959x over previous
import functools

import jax
import jax.numpy as jnp
from jax import lax
from jax.experimental import pallas as pl
from jax.experimental.pallas import tpu as pltpu
from jax.experimental.pallas import tpu_sc as plsc

_BATCH = 16384
_L = 50
_EMB = 128
_TOT = _BATCH * _L
_NW = 32
_PER_W = _TOT // _NW
_G = 128
_NCH = _PER_W // _G
_NBUF = 4


def _make_gather():
    mesh = plsc.VectorSubcoreMesh(core_axis_name="c", subcore_axis_name="s")

    @functools.partial(
        pl.kernel,
        out_type=jax.ShapeDtypeStruct((_TOT, _EMB), jnp.float32),
        mesh=mesh,
        scratch_types=[
            pltpu.VMEM((_NCH, _G), jnp.int32),
            pltpu.VMEM((_NBUF, _G, _EMB), jnp.float32),
        ] + [pltpu.SemaphoreType.DMA] * (2 * _NBUF),
    )
    def gather_kernel(idx_hbm, table_hbm, out_hbm, idx_v, rows_v, *sems):
        gsem = sems[:_NBUF]
        ssem = sems[_NBUF:]
        wid = lax.axis_index("s") * 2 + lax.axis_index("c")
        base = wid * _PER_W
        pltpu.sync_copy(idx_hbm.at[wid], idx_v)

        def start_gather(j, b):
            pltpu.async_copy(table_hbm.at[idx_v.at[j]], rows_v.at[b], gsem[b])

        def start_store(j, b):
            pltpu.async_copy(
                rows_v.at[b], out_hbm.at[pl.ds(base + j * _G, _G)], ssem[b])

        def wait_gather(b):
            pltpu.make_async_copy(
                table_hbm.at[pl.ds(0, _G)], rows_v.at[b], gsem[b]).wait()

        def wait_store(j, b):
            pltpu.make_async_copy(
                rows_v.at[b], out_hbm.at[pl.ds(base + j * _G, _G)],
                ssem[b]).wait()

        for b in range(_NBUF):
            start_gather(b, b)

        def body(t, carry):
            j0 = t * _NBUF
            for b in range(_NBUF):
                wait_gather(b)
                start_store(j0 + b, b)
            for b in range(_NBUF):
                wait_store(j0 + b, b)
                start_gather(j0 + b + _NBUF, b)
            return carry

        lax.fori_loop(0, _NCH // _NBUF - 1, body, 0)

        j0 = _NCH - _NBUF
        for b in range(_NBUF):
            wait_gather(b)
            start_store(j0 + b, b)
        for b in range(_NBUF):
            wait_store(j0 + b, b)

    return gather_kernel


_gather = _make_gather()

def kernel(data, ivectors_weight):
    idx = data.astype(jnp.int32).T.reshape(_NW, _NCH, _G)
    flat = _gather(idx, ivectors_weight)
    return flat.reshape(_L, _BATCH, _EMB).transpose(1, 0, 2)


# speedup vs baseline: 15.8220x; 1.0004x over previous
import functools

import jax
import jax.numpy as jnp
from jax import lax
from jax.experimental import pallas as pl
from jax.experimental.pallas import tpu as pltpu
from jax.experimental.pallas import tpu_sc as plsc

_BATCH = 16384
_L = 50
_EMB = 128
_TOT = _BATCH * _L
_NW = 32
_PER_W = _TOT // _NW
_G = 128
_NCH = _PER_W // _G
_NBUF = 5


def _make_gather():
    mesh = plsc.VectorSubcoreMesh(core_axis_name="c", subcore_axis_name="s")

    @functools.partial(
        pl.kernel,
        out_type=jax.ShapeDtypeStruct((_TOT, _EMB), jnp.float32),
        mesh=mesh,
        scratch_types=[
            pltpu.VMEM((_NCH, _G), jnp.int32),
            pltpu.VMEM((_NBUF, _G, _EMB), jnp.float32),
        ] + [pltpu.SemaphoreType.DMA] * (2 * _NBUF),
    )
    def gather_kernel(idx_hbm, table_hbm, out_hbm, idx_v, rows_v, *sems):
        gsem = sems[:_NBUF]
        ssem = sems[_NBUF:]
        wid = lax.axis_index("s") * 2 + lax.axis_index("c")
        base = wid * _PER_W
        pltpu.sync_copy(idx_hbm.at[wid], idx_v)

        def start_gather(j, b):
            pltpu.async_copy(table_hbm.at[idx_v.at[j]], rows_v.at[b], gsem[b])

        def start_store(j, b):
            pltpu.async_copy(
                rows_v.at[b], out_hbm.at[pl.ds(base + j * _G, _G)], ssem[b])

        def wait_gather(b):
            pltpu.make_async_copy(
                table_hbm.at[pl.ds(0, _G)], rows_v.at[b], gsem[b]).wait()

        def wait_store(j, b):
            pltpu.make_async_copy(
                rows_v.at[b], out_hbm.at[pl.ds(base + j * _G, _G)],
                ssem[b]).wait()

        for b in range(_NBUF):
            start_gather(b, b)

        def body(t, carry):
            j0 = t * _NBUF
            for b in range(_NBUF):
                wait_gather(b)
                start_store(j0 + b, b)
            for b in range(_NBUF):
                wait_store(j0 + b, b)
                start_gather(j0 + b + _NBUF, b)
            return carry

        lax.fori_loop(0, _NCH // _NBUF - 1, body, 0)

        j0 = _NCH - _NBUF
        for b in range(_NBUF):
            wait_gather(b)
            start_store(j0 + b, b)
        for b in range(_NBUF):
            wait_store(j0 + b, b)

    return gather_kernel


_gather = _make_gather()

def kernel(data, ivectors_weight):
    idx = data.astype(jnp.int32).T.reshape(_NW, _NCH, _G)
    flat = _gather(idx, ivectors_weight)
    return flat.reshape(_L, _BATCH, _EMB).transpose(1, 0, 2)
